# baseline (speedup 1.0000x reference)
import functools

import jax
import jax.numpy as jnp
from jax import lax
from jax.experimental import pallas as pl
from jax.experimental.pallas import tpu as pltpu
from jax.experimental.pallas import tpu_sc as plsc

N_FACTORS = 64
BATCH = 16384
NC = 2
NS = 16
NW = NC * NS
B_PER_W = BATCH // NW
ROWS_PER_DMA = 128
N_DMA = B_PER_W // ROWS_PER_DMA
GROUPS = B_PER_W // 16


def _body(uidx_hbm, midx_hbm, ut_hbm, mt_hbm, w_hbm, b_hbm, out_hbm,
          uidx_v, midx_v, urows_v, mrows_v, w_v, b_v, out_v, sem):
    wid = lax.axis_index("s") * NC + lax.axis_index("c")
    base = wid * N_DMA

    pltpu.sync_copy(w_hbm, w_v)
    pltpu.sync_copy(b_hbm, b_v)
    pltpu.sync_copy(uidx_hbm.at[pl.ds(base, N_DMA)], uidx_v)
    pltpu.sync_copy(midx_hbm.at[pl.ds(base, N_DMA)], midx_v)

    copies = []
    for j in range(N_DMA):
        dst = pl.ds(j * ROWS_PER_DMA, ROWS_PER_DMA)
        copies.append(pltpu.async_copy(ut_hbm.at[uidx_v.at[j]],
                                       urows_v.at[dst], sem))
        copies.append(pltpu.async_copy(mt_hbm.at[midx_v.at[j]],
                                       mrows_v.at[dst], sem))
    for cp in copies:
        cp.wait()

    wchunks = [w_v[pl.ds(k * 16, 16)] for k in range(2 * N_FACTORS // 16)]

    def group(g, carry):
        rid = g * 16 + lax.iota(jnp.int32, 16)
        acc = b_v[...]
        for c in range(N_FACTORS):
            cc = jnp.full((16,), c, jnp.int32)
            lane = jnp.full((16,), c % 16, jnp.int32)
            wu = wchunks[c // 16].at[lane].get(mode="promise_in_bounds")
            wm = wchunks[N_FACTORS // 16 + c // 16].at[lane].get(
                mode="promise_in_bounds")
            ucol = plsc.load_gather(urows_v, [rid, cc])
            mcol = plsc.load_gather(mrows_v, [rid, cc])
            acc = acc + ucol * wu + mcol * wm
        out_v[pl.ds(g * 16, 16)] = acc
        return carry

    lax.fori_loop(0, GROUPS, group, 0)

    pltpu.sync_copy(out_v, out_hbm.at[pl.ds(wid * B_PER_W, B_PER_W)])


@functools.partial(jax.jit, static_argnames=())
def _run(uidx, midx, ut, mt, w, bvec):
    mesh = plsc.VectorSubcoreMesh(core_axis_name="c", subcore_axis_name="s",
                                  num_cores=NC, num_subcores=NS)
    fn = pl.kernel(
        _body,
        out_type=jax.ShapeDtypeStruct((BATCH,), jnp.float32),
        mesh=mesh,
        compiler_params=pltpu.CompilerParams(needs_layout_passes=False,
                                             use_tc_tiling_on_sc=False),
        scratch_types=[
            pltpu.VMEM((N_DMA, ROWS_PER_DMA), jnp.int32),
            pltpu.VMEM((N_DMA, ROWS_PER_DMA), jnp.int32),
            pltpu.VMEM((B_PER_W, N_FACTORS), jnp.float32),
            pltpu.VMEM((B_PER_W, N_FACTORS), jnp.float32),
            pltpu.VMEM((2 * N_FACTORS,), jnp.float32),
            pltpu.VMEM((16,), jnp.float32),
            pltpu.VMEM((B_PER_W,), jnp.float32),
            pltpu.SemaphoreType.DMA,
        ],
    )
    return fn(uidx, midx, ut, mt, w, bvec)


def kernel(user, movie, user_table, movie_table, W, b):
    uidx = user.astype(jnp.int32).reshape(NW * N_DMA, ROWS_PER_DMA)
    midx = movie.astype(jnp.int32).reshape(NW * N_DMA, ROWS_PER_DMA)
    w = W.reshape(2 * N_FACTORS)
    bvec = jnp.broadcast_to(b, (16,))
    out = _run(uidx, midx, user_table, movie_table, w, bvec)
    return out.reshape(BATCH, 1)
